# Initial kernel scaffold; baseline (speedup 1.0000x reference)
import jax
import jax.numpy as jnp
from jax.experimental import pallas as pl


def kernel(low, cur, high, num_features):
    raise NotImplementedError("write your pallas kernel here")


# speedup vs baseline: 1.7097x; 1.7097x over previous
import functools

import jax
import jax.numpy as jnp
from jax import lax
from jax.experimental import pallas as pl

_H = 2048
_W = 2048
_BLK = 64
_GRID = _H // _BLK
_K = 2000
_EPS_NMS = 1e-5
_EPS_DEN = 1e-8


def _nms_body(lo_ref, cu_ref, hi_ref,
              lo_u, cu_u, hi_u, lo_d, cu_d, hi_d,
              nm_ref, den_ref, ns_ref, ny_ref, nx_ref):
    i = pl.program_id(0)
    cu_blk = cu_ref[:]

    mp = None
    den = None
    ns = None
    ny = None
    nx = None
    planes = (
        (lo_ref, lo_u, lo_d, -1.0),
        (cu_ref, cu_u, cu_d, 0.0),
        (hi_ref, hi_u, hi_d, 1.0),
    )
    for (ref, uref, dref, zc) in planes:
        full = jnp.concatenate([uref[0], ref[:], dref[0]], axis=0)
        for dy in (-1, 0, 1):
            base = lax.slice_in_dim(full, dy + 1, dy + 1 + _BLK, axis=0)
            for dx in (-1, 0, 1):
                v = base if dx == 0 else jnp.roll(base, -dx, axis=1)
                mp = v if mp is None else jnp.maximum(mp, v)
                den = v if den is None else den + v
                if zc != 0.0:
                    ns = zc * v if ns is None else ns + zc * v
                if dy != 0:
                    ny = float(dy) * v if ny is None else ny + float(dy) * v
                if dx != 0:
                    nx = float(dx) * v if nx is None else nx + float(dx) * v

    col = lax.broadcasted_iota(jnp.int32, (_BLK, _W), 1)
    row = lax.broadcasted_iota(jnp.int32, (_BLK, _W), 0) + i * _BLK
    keep = (cu_blk - mp + _EPS_NMS > 0)
    keep = jnp.logical_and(keep, jnp.logical_and(col > 0, col < _W - 1))
    keep = jnp.logical_and(keep, jnp.logical_and(row > 0, row < _H - 1))
    nm_ref[:] = jnp.where(keep, cu_blk, 0.0)
    den_ref[:] = den
    ns_ref[:] = ns
    ny_ref[:] = ny
    nx_ref[:] = nx


def _edges(x):
    zero = jnp.zeros((1, _W), x.dtype)
    up = jnp.concatenate([zero, x[_BLK - 1::_BLK][: _GRID - 1]], axis=0)
    down = jnp.concatenate([x[_BLK::_BLK], zero], axis=0)
    return up.reshape(_GRID, 1, _W), down.reshape(_GRID, 1, _W)


@functools.partial(jax.jit, static_argnums=())
def _run(low, cur, high):
    lo = low.reshape(_H, _W)
    cu = cur.reshape(_H, _W)
    hi = high.reshape(_H, _W)
    lo_u, lo_d = _edges(lo)
    cu_u, cu_d = _edges(cu)
    hi_u, hi_d = _edges(hi)

    blk = pl.BlockSpec((_BLK, _W), lambda i: (i, 0))
    eblk = pl.BlockSpec((1, 1, _W), lambda i: (i, 0, 0))
    out_sd = jax.ShapeDtypeStruct((_H, _W), jnp.float32)
    nm, den, ns, ny, nx = pl.pallas_call(
        _nms_body,
        grid=(_GRID,),
        in_specs=[blk, blk, blk, eblk, eblk, eblk, eblk, eblk, eblk],
        out_specs=[blk, blk, blk, blk, blk],
        out_shape=[out_sd, out_sd, out_sd, out_sd, out_sd],
    )(lo, cu, hi, lo_u, cu_u, hi_u, lo_d, cu_d, hi_d)

    vals, idxs = lax.top_k(nm.reshape(-1), _K)

    den_g = den.reshape(-1)[idxs] + _EPS_DEN
    s = ns.reshape(-1)[idxs] / den_g
    yy = ny.reshape(-1)[idxs] / den_g
    xx = nx.reshape(-1)[idxs] / den_g
    yf = (idxs // _W).astype(jnp.float32)
    xf = (idxs % _W).astype(jnp.float32)
    inv = 1.0 / float(_W)
    s = s * inv
    yc = (yy + yf) * inv
    xc = (xx + xf) * inv
    zeros = jnp.zeros_like(s)
    row0 = jnp.stack([s, zeros, xc], axis=1)
    row1 = jnp.stack([zeros, s, yc], axis=1)
    lafs = jnp.stack([row0, row1], axis=1)
    return vals, lafs


def kernel(low, cur, high, num_features):
    vals, lafs = _run(low, cur, high)
    return vals, lafs


# speedup vs baseline: 16.6908x; 9.7623x over previous
import functools

import jax
import jax.numpy as jnp
from jax import lax
from jax.experimental import pallas as pl
from jax.experimental.pallas import tpu as pltpu
from jax.experimental.pallas import tpu_sc as plsc

_H = 2048
_W = 2048
_BLK = 64
_GRID = _H // _BLK
_K = 2000
_EPS_NMS = 1e-5
_EPS_DEN = 1e-8

_NW = 32
_WROWS = _H // _NW
_CAND = 16400
_SLOT = 2048
_OBUF = _SLOT + 32


def _nms_body(lo_ref, cu_ref, hi_ref,
              lo_u, cu_u, hi_u, lo_d, cu_d, hi_d,
              nm_ref, den_ref, ns_ref, ny_ref, nx_ref):
    i = pl.program_id(0)
    cu_blk = cu_ref[:]

    mp = None
    den = None
    ns = None
    ny = None
    nx = None
    planes = (
        (lo_ref, lo_u, lo_d, -1.0),
        (cu_ref, cu_u, cu_d, 0.0),
        (hi_ref, hi_u, hi_d, 1.0),
    )
    for (ref, uref, dref, zc) in planes:
        full = jnp.concatenate([uref[0], ref[:], dref[0]], axis=0)
        for dy in (-1, 0, 1):
            base = lax.slice_in_dim(full, dy + 1, dy + 1 + _BLK, axis=0)
            for dx in (-1, 0, 1):
                v = base if dx == 0 else jnp.roll(base, -dx, axis=1)
                mp = v if mp is None else jnp.maximum(mp, v)
                den = v if den is None else den + v
                if zc != 0.0:
                    ns = zc * v if ns is None else ns + zc * v
                if dy != 0:
                    ny = float(dy) * v if ny is None else ny + float(dy) * v
                if dx != 0:
                    nx = float(dx) * v if nx is None else nx + float(dx) * v

    col = lax.broadcasted_iota(jnp.int32, (_BLK, _W), 1)
    row = lax.broadcasted_iota(jnp.int32, (_BLK, _W), 0) + i * _BLK
    keep = (cu_blk - mp + _EPS_NMS > 0)
    keep = jnp.logical_and(keep, jnp.logical_and(col > 0, col < _W - 1))
    keep = jnp.logical_and(keep, jnp.logical_and(row > 0, row < _H - 1))
    nm_ref[:] = jnp.where(keep, cu_blk, 0.0)
    den_ref[:] = den
    ns_ref[:] = ns
    ny_ref[:] = ny
    nx_ref[:] = nx


def _popcnt(m):
    return jnp.max(plsc.all_reduce_population_count(m))


def _select_body(nm_hbm, ov_hbm, oi_hbm,
                 chunk_v, cand_v, cand_i, outv_v, outi_v):
    cid = lax.axis_index("c")
    sid = lax.axis_index("s")
    wid = cid * 16 + sid

    neg1 = jnp.full((16,), -1.0, jnp.float32)
    zero_i = jnp.zeros((16,), jnp.int32)

    def fill(k, _):
        cand_v[pl.ds(k * 16, 16)] = neg1
        return 0
    lax.fori_loop(0, _CAND // 16, fill, 0)

    cnt = jnp.int32(0)
    lanes = lax.iota(jnp.int32, 16)
    for c in range(_WROWS // 8):
        row0 = wid * _WROWS + c * 8
        pltpu.sync_copy(nm_hbm.at[pl.ds(row0, 8)], chunk_v)

        def scan_body(j, cnt, c=c, row0=row0):
            r = j // 128
            col = (j % 128) * 16
            v = chunk_v[r, pl.ds(col, 16)]
            m = v > 0.0
            base = (row0 + r) * _W + col
            idxv = jnp.full((16,), base, jnp.int32) + lanes
            inc = m.astype(jnp.int32)
            pref = plsc.cumsum(inc)
            pos = jnp.minimum(cnt, _CAND - 48) + pref - 1
            pos = jnp.where(m, pos, _CAND - 16 + lanes)
            plsc.store_scatter(cand_v, [pos], v)
            plsc.store_scatter(cand_i, [pos], idxv)
            return cnt + jnp.max(pref)
        cnt = lax.fori_loop(0, 1024, scan_body, cnt)

    nvec = (cnt + 15) // 16

    def count_ge(tv):
        def cbody(j, acc):
            v = cand_v[pl.ds(j * 16, 16)]
            return acc + _popcnt(v >= tv)
        return lax.fori_loop(0, nvec, cbody, jnp.int32(0))

    target = jnp.minimum(jnp.int32(_K), cnt)

    def round_body(it, carry):
        lo, hi = carry
        mid = (lo + hi) // 2
        tv = lax.bitcast_convert_type(jnp.full((16,), mid, jnp.int32),
                                      jnp.float32)
        ge = count_ge(tv) >= target
        lo = jnp.where(ge, mid, lo)
        hi = jnp.where(ge, hi, mid)
        return lo, hi

    lo, hi = lax.fori_loop(0, 30, round_body,
                           (jnp.int32(0), jnp.int32(0x3F800000)))
    tv = lax.bitcast_convert_type(jnp.full((16,), lo, jnp.int32), jnp.float32)

    def ofill(k, _):
        outv_v[pl.ds(k * 16, 16)] = neg1
        outi_v[pl.ds(k * 16, 16)] = zero_i
        return 0
    lax.fori_loop(0, _OBUF // 16, ofill, 0)

    def ebody(j, ocnt):
        off = jnp.minimum(ocnt, _SLOT)
        v = cand_v[pl.ds(j * 16, 16)]
        iv = cand_i[pl.ds(j * 16, 16)]
        m = v >= tv
        pref = plsc.cumsum(m.astype(jnp.int32))
        pos = off + pref - 1
        pos = jnp.where(m, pos, _SLOT + 16 + lanes)
        plsc.store_scatter(outv_v, [pos], v)
        plsc.store_scatter(outi_v, [pos], iv)
        return off + jnp.max(pref)
    lax.fori_loop(0, nvec, ebody, jnp.int32(0))

    pltpu.sync_copy(outv_v.at[pl.ds(0, _SLOT)], ov_hbm.at[wid])
    pltpu.sync_copy(outi_v.at[pl.ds(0, _SLOT)], oi_hbm.at[wid])


def _select_topk(nm):
    mesh = plsc.VectorSubcoreMesh(core_axis_name="c", subcore_axis_name="s")
    sel = pl.kernel(
        _select_body,
        mesh=mesh,
        out_type=[
            jax.ShapeDtypeStruct((_NW, _SLOT), jnp.float32),
            jax.ShapeDtypeStruct((_NW, _SLOT), jnp.int32),
        ],
        scratch_types=[
            pltpu.VMEM((8, _W), jnp.float32),
            pltpu.VMEM((_CAND,), jnp.float32),
            pltpu.VMEM((_CAND,), jnp.int32),
            pltpu.VMEM((_OBUF,), jnp.float32),
            pltpu.VMEM((_OBUF,), jnp.int32),
        ],
        compiler_params=pltpu.CompilerParams(needs_layout_passes=False),
    )
    ov, oi = sel(nm)
    vals, pos = lax.top_k(ov.reshape(-1), _K)
    idxs = oi.reshape(-1)[pos]
    return vals, idxs


def _edges(x):
    zero = jnp.zeros((1, _W), x.dtype)
    up = jnp.concatenate([zero, x[_BLK - 1::_BLK][: _GRID - 1]], axis=0)
    down = jnp.concatenate([x[_BLK::_BLK], zero], axis=0)
    return up.reshape(_GRID, 1, _W), down.reshape(_GRID, 1, _W)


@functools.partial(jax.jit, static_argnums=())
def _run(low, cur, high):
    lo = low.reshape(_H, _W)
    cu = cur.reshape(_H, _W)
    hi = high.reshape(_H, _W)
    lo_u, lo_d = _edges(lo)
    cu_u, cu_d = _edges(cu)
    hi_u, hi_d = _edges(hi)

    blk = pl.BlockSpec((_BLK, _W), lambda i: (i, 0))
    eblk = pl.BlockSpec((1, 1, _W), lambda i: (i, 0, 0))
    out_sd = jax.ShapeDtypeStruct((_H, _W), jnp.float32)
    nm, den, ns, ny, nx = pl.pallas_call(
        _nms_body,
        grid=(_GRID,),
        in_specs=[blk, blk, blk, eblk, eblk, eblk, eblk, eblk, eblk],
        out_specs=[blk, blk, blk, blk, blk],
        out_shape=[out_sd, out_sd, out_sd, out_sd, out_sd],
    )(lo, cu, hi, lo_u, cu_u, hi_u, lo_d, cu_d, hi_d)

    vals, idxs = _select_topk(nm)

    den_g = den.reshape(-1)[idxs] + _EPS_DEN
    s = ns.reshape(-1)[idxs] / den_g
    yy = ny.reshape(-1)[idxs] / den_g
    xx = nx.reshape(-1)[idxs] / den_g
    yf = (idxs // _W).astype(jnp.float32)
    xf = (idxs % _W).astype(jnp.float32)
    inv = 1.0 / float(_W)
    s = s * inv
    yc = (yy + yf) * inv
    xc = (xx + xf) * inv
    zeros = jnp.zeros_like(s)
    row0 = jnp.stack([s, zeros, xc], axis=1)
    row1 = jnp.stack([zeros, s, yc], axis=1)
    lafs = jnp.stack([row0, row1], axis=1)
    return vals, lafs


def kernel(low, cur, high, num_features):
    vals, lafs = _run(low, cur, high)
    return vals, lafs


# speedup vs baseline: 16.8327x; 1.0085x over previous
import functools

import jax
import jax.numpy as jnp
from jax import lax
from jax.experimental import pallas as pl
from jax.experimental.pallas import tpu as pltpu
from jax.experimental.pallas import tpu_sc as plsc

_H = 2048
_W = 2048
_BLK = 64
_GRID = _H // _BLK
_K = 2000
_EPS_NMS = 1e-5
_EPS_DEN = 1e-8

_NW = 32
_WROWS = _H // _NW
_CAND = 16400
_SLOT = 2048
_OBUF = _SLOT + 32


def _nms_body(lo_ref, cu_ref, hi_ref,
              lo_u, cu_u, hi_u, lo_d, cu_d, hi_d,
              nm_ref, den_ref, ns_ref, ny_ref, nx_ref):
    i = pl.program_id(0)
    cu_blk = cu_ref[:]

    mp = None
    den = None
    ns = None
    ny = None
    nx = None
    planes = (
        (lo_ref, lo_u, lo_d, -1.0),
        (cu_ref, cu_u, cu_d, 0.0),
        (hi_ref, hi_u, hi_d, 1.0),
    )
    for (ref, uref, dref, zc) in planes:
        full = jnp.concatenate([uref[0], ref[:], dref[0]], axis=0)
        for dy in (-1, 0, 1):
            base = lax.slice_in_dim(full, dy + 1, dy + 1 + _BLK, axis=0)
            for dx in (-1, 0, 1):
                v = base if dx == 0 else jnp.roll(base, -dx, axis=1)
                mp = v if mp is None else jnp.maximum(mp, v)
                den = v if den is None else den + v
                if zc != 0.0:
                    ns = zc * v if ns is None else ns + zc * v
                if dy != 0:
                    ny = float(dy) * v if ny is None else ny + float(dy) * v
                if dx != 0:
                    nx = float(dx) * v if nx is None else nx + float(dx) * v

    col = lax.broadcasted_iota(jnp.int32, (_BLK, _W), 1)
    row = lax.broadcasted_iota(jnp.int32, (_BLK, _W), 0) + i * _BLK
    keep = (cu_blk - mp + _EPS_NMS > 0)
    keep = jnp.logical_and(keep, jnp.logical_and(col > 0, col < _W - 1))
    keep = jnp.logical_and(keep, jnp.logical_and(row > 0, row < _H - 1))
    nm_ref[:] = jnp.where(keep, cu_blk, 0.0)
    den_ref[:] = den
    ns_ref[:] = ns
    ny_ref[:] = ny
    nx_ref[:] = nx


def _popcnt(m):
    return jnp.max(plsc.all_reduce_population_count(m))


def _select_body(nm_hbm, ov_hbm, oi_hbm,
                 chunk_v, cand_v, cand_i, outv_v, outi_v):
    cid = lax.axis_index("c")
    sid = lax.axis_index("s")
    wid = cid * 16 + sid

    neg1 = jnp.full((16,), -1.0, jnp.float32)
    zero_i = jnp.zeros((16,), jnp.int32)

    def fill(k, _):
        cand_v[pl.ds(k * 16, 16)] = neg1
        return 0
    lax.fori_loop(0, _CAND // 16, fill, 0)

    cnt = jnp.int32(0)
    lanes = lax.iota(jnp.int32, 16)
    for c in range(_WROWS // 8):
        row0 = wid * _WROWS + c * 8
        pltpu.sync_copy(nm_hbm.at[pl.ds(row0, 8)], chunk_v)

        def scan_body(j, cnt, c=c, row0=row0):
            r = j // 128
            col = (j % 128) * 16
            v = chunk_v[r, pl.ds(col, 16)]
            m = v > 0.0
            base = (row0 + r) * _W + col
            idxv = jnp.full((16,), base, jnp.int32) + lanes
            inc = m.astype(jnp.int32)
            pref = plsc.cumsum(inc)
            pos = jnp.minimum(cnt, _CAND - 48) + pref - 1
            pos = jnp.where(m, pos, _CAND - 16 + lanes)
            plsc.store_scatter(cand_v, [pos], v)
            plsc.store_scatter(cand_i, [pos], idxv)
            return cnt + pref[15]
        cnt = lax.fori_loop(0, 1024, scan_body, cnt)

    nvec = (cnt + 15) // 16

    def count_ge(tv):
        def cbody(j, acc):
            v = cand_v[pl.ds(j * 16, 16)]
            pref = plsc.cumsum((v >= tv).astype(jnp.int32))
            return acc + pref[15]
        return lax.fori_loop(0, nvec, cbody, jnp.int32(0))

    target = jnp.minimum(jnp.int32(_K), cnt)

    def round_body(it, carry):
        lo, hi = carry
        mid = (lo + hi) // 2
        tv = lax.bitcast_convert_type(jnp.full((16,), mid, jnp.int32),
                                      jnp.float32)
        ge = count_ge(tv) >= target
        lo = jnp.where(ge, mid, lo)
        hi = jnp.where(ge, hi, mid)
        return lo, hi

    lo, hi = lax.fori_loop(0, 30, round_body,
                           (jnp.int32(0), jnp.int32(0x3F800000)))
    tv = lax.bitcast_convert_type(jnp.full((16,), lo, jnp.int32), jnp.float32)

    def ofill(k, _):
        outv_v[pl.ds(k * 16, 16)] = neg1
        outi_v[pl.ds(k * 16, 16)] = zero_i
        return 0
    lax.fori_loop(0, _OBUF // 16, ofill, 0)

    def ebody(j, ocnt):
        off = jnp.minimum(ocnt, _SLOT)
        v = cand_v[pl.ds(j * 16, 16)]
        iv = cand_i[pl.ds(j * 16, 16)]
        m = v >= tv
        pref = plsc.cumsum(m.astype(jnp.int32))
        pos = off + pref - 1
        pos = jnp.where(m, pos, _SLOT + 16 + lanes)
        plsc.store_scatter(outv_v, [pos], v)
        plsc.store_scatter(outi_v, [pos], iv)
        return off + pref[15]
    lax.fori_loop(0, nvec, ebody, jnp.int32(0))

    pltpu.sync_copy(outv_v.at[pl.ds(0, _SLOT)], ov_hbm.at[wid])
    pltpu.sync_copy(outi_v.at[pl.ds(0, _SLOT)], oi_hbm.at[wid])


def _select_topk(nm):
    mesh = plsc.VectorSubcoreMesh(core_axis_name="c", subcore_axis_name="s")
    sel = pl.kernel(
        _select_body,
        mesh=mesh,
        out_type=[
            jax.ShapeDtypeStruct((_NW, _SLOT), jnp.float32),
            jax.ShapeDtypeStruct((_NW, _SLOT), jnp.int32),
        ],
        scratch_types=[
            pltpu.VMEM((8, _W), jnp.float32),
            pltpu.VMEM((_CAND,), jnp.float32),
            pltpu.VMEM((_CAND,), jnp.int32),
            pltpu.VMEM((_OBUF,), jnp.float32),
            pltpu.VMEM((_OBUF,), jnp.int32),
        ],
        compiler_params=pltpu.CompilerParams(needs_layout_passes=False),
    )
    ov, oi = sel(nm)
    vals, pos = lax.top_k(ov.reshape(-1), _K)
    idxs = oi.reshape(-1)[pos]
    return vals, idxs


def _edges(x):
    zero = jnp.zeros((1, _W), x.dtype)
    up = jnp.concatenate([zero, x[_BLK - 1::_BLK][: _GRID - 1]], axis=0)
    down = jnp.concatenate([x[_BLK::_BLK], zero], axis=0)
    return up.reshape(_GRID, 1, _W), down.reshape(_GRID, 1, _W)


@functools.partial(jax.jit, static_argnums=())
def _run(low, cur, high):
    lo = low.reshape(_H, _W)
    cu = cur.reshape(_H, _W)
    hi = high.reshape(_H, _W)
    lo_u, lo_d = _edges(lo)
    cu_u, cu_d = _edges(cu)
    hi_u, hi_d = _edges(hi)

    blk = pl.BlockSpec((_BLK, _W), lambda i: (i, 0))
    eblk = pl.BlockSpec((1, 1, _W), lambda i: (i, 0, 0))
    out_sd = jax.ShapeDtypeStruct((_H, _W), jnp.float32)
    nm, den, ns, ny, nx = pl.pallas_call(
        _nms_body,
        grid=(_GRID,),
        in_specs=[blk, blk, blk, eblk, eblk, eblk, eblk, eblk, eblk],
        out_specs=[blk, blk, blk, blk, blk],
        out_shape=[out_sd, out_sd, out_sd, out_sd, out_sd],
    )(lo, cu, hi, lo_u, cu_u, hi_u, lo_d, cu_d, hi_d)

    vals, idxs = _select_topk(nm)

    den_g = den.reshape(-1)[idxs] + _EPS_DEN
    s = ns.reshape(-1)[idxs] / den_g
    yy = ny.reshape(-1)[idxs] / den_g
    xx = nx.reshape(-1)[idxs] / den_g
    yf = (idxs // _W).astype(jnp.float32)
    xf = (idxs % _W).astype(jnp.float32)
    inv = 1.0 / float(_W)
    s = s * inv
    yc = (yy + yf) * inv
    xc = (xx + xf) * inv
    zeros = jnp.zeros_like(s)
    row0 = jnp.stack([s, zeros, xc], axis=1)
    row1 = jnp.stack([zeros, s, yc], axis=1)
    lafs = jnp.stack([row0, row1], axis=1)
    return vals, lafs


def kernel(low, cur, high, num_features):
    vals, lafs = _run(low, cur, high)
    return vals, lafs


# speedup vs baseline: 20.5701x; 1.2220x over previous
import functools

import jax
import jax.numpy as jnp
from jax import lax
from jax.experimental import pallas as pl
from jax.experimental.pallas import tpu as pltpu
from jax.experimental.pallas import tpu_sc as plsc

_H = 2048
_W = 2048
_BLK = 64
_GRID = _H // _BLK
_K = 2000
_EPS_NMS = 1e-5
_EPS_DEN = 1e-8

_NW = 32
_WROWS = _H // _NW
_CAND = 16400
_SLOT = 2048
_OBUF = _SLOT + 32


def _nms_body(lo_ref, cu_ref, hi_ref,
              lo_u, cu_u, hi_u, lo_d, cu_d, hi_d, nm_ref):
    i = pl.program_id(0)
    cu_blk = cu_ref[:]

    mp = None
    planes = (
        (lo_ref, lo_u, lo_d),
        (cu_ref, cu_u, cu_d),
        (hi_ref, hi_u, hi_d),
    )
    for (ref, uref, dref) in planes:
        full = jnp.concatenate([uref[0], ref[:], dref[0]], axis=0)
        for dy in (-1, 0, 1):
            base = lax.slice_in_dim(full, dy + 1, dy + 1 + _BLK, axis=0)
            for dx in (-1, 0, 1):
                v = base if dx == 0 else jnp.roll(base, -dx, axis=1)
                mp = v if mp is None else jnp.maximum(mp, v)

    col = lax.broadcasted_iota(jnp.int32, (_BLK, _W), 1)
    row = lax.broadcasted_iota(jnp.int32, (_BLK, _W), 0) + i * _BLK
    keep = (cu_blk - mp + _EPS_NMS > 0)
    keep = jnp.logical_and(keep, jnp.logical_and(col > 0, col < _W - 1))
    keep = jnp.logical_and(keep, jnp.logical_and(row > 0, row < _H - 1))
    nm_ref[:] = jnp.where(keep, cu_blk, 0.0)


def _popcnt(m):
    return jnp.max(plsc.all_reduce_population_count(m))


def _select_body(nm_hbm, ov_hbm, oi_hbm,
                 chunk_v, cand_v, cand_i, outv_v, outi_v):
    cid = lax.axis_index("c")
    sid = lax.axis_index("s")
    wid = cid * 16 + sid

    neg1 = jnp.full((16,), -1.0, jnp.float32)
    zero_i = jnp.zeros((16,), jnp.int32)

    def fill(k, _):
        cand_v[pl.ds(k * 16, 16)] = neg1
        return 0
    lax.fori_loop(0, _CAND // 16, fill, 0)

    cnt = jnp.int32(0)
    lanes = lax.iota(jnp.int32, 16)
    for c in range(_WROWS // 8):
        row0 = wid * _WROWS + c * 8
        pltpu.sync_copy(nm_hbm.at[pl.ds(row0, 8)], chunk_v)

        def scan_body(j, cnt, c=c, row0=row0):
            r = j // 128
            col = (j % 128) * 16
            v = chunk_v[r, pl.ds(col, 16)]
            m = v > 0.0
            base = (row0 + r) * _W + col
            idxv = jnp.full((16,), base, jnp.int32) + lanes
            inc = m.astype(jnp.int32)
            pref = plsc.cumsum(inc)
            pos = jnp.minimum(cnt, _CAND - 48) + pref - 1
            pos = jnp.where(m, pos, _CAND - 16 + lanes)
            plsc.store_scatter(cand_v, [pos], v)
            plsc.store_scatter(cand_i, [pos], idxv)
            return cnt + pref[15]
        cnt = lax.fori_loop(0, 1024, scan_body, cnt)

    nvec = (cnt + 15) // 16

    def count_ge(tv):
        def cbody(j, acc):
            v = cand_v[pl.ds(j * 16, 16)]
            pref = plsc.cumsum((v >= tv).astype(jnp.int32))
            return acc + pref[15]
        return lax.fori_loop(0, nvec, cbody, jnp.int32(0))

    target = jnp.minimum(jnp.int32(_K), cnt)

    def round_body(it, carry):
        lo, hi = carry
        mid = (lo + hi) // 2
        tv = lax.bitcast_convert_type(jnp.full((16,), mid, jnp.int32),
                                      jnp.float32)
        ge = count_ge(tv) >= target
        lo = jnp.where(ge, mid, lo)
        hi = jnp.where(ge, hi, mid)
        return lo, hi

    lo, hi = lax.fori_loop(0, 30, round_body,
                           (jnp.int32(0), jnp.int32(0x3F800000)))
    tv = lax.bitcast_convert_type(jnp.full((16,), lo, jnp.int32), jnp.float32)

    def ofill(k, _):
        outv_v[pl.ds(k * 16, 16)] = neg1
        outi_v[pl.ds(k * 16, 16)] = zero_i
        return 0
    lax.fori_loop(0, _OBUF // 16, ofill, 0)

    def ebody(j, ocnt):
        off = jnp.minimum(ocnt, _SLOT)
        v = cand_v[pl.ds(j * 16, 16)]
        iv = cand_i[pl.ds(j * 16, 16)]
        m = v >= tv
        pref = plsc.cumsum(m.astype(jnp.int32))
        pos = off + pref - 1
        pos = jnp.where(m, pos, _SLOT + 16 + lanes)
        plsc.store_scatter(outv_v, [pos], v)
        plsc.store_scatter(outi_v, [pos], iv)
        return off + pref[15]
    lax.fori_loop(0, nvec, ebody, jnp.int32(0))

    pltpu.sync_copy(outv_v.at[pl.ds(0, _SLOT)], ov_hbm.at[wid])
    pltpu.sync_copy(outi_v.at[pl.ds(0, _SLOT)], oi_hbm.at[wid])


def _select_topk(nm):
    mesh = plsc.VectorSubcoreMesh(core_axis_name="c", subcore_axis_name="s")
    sel = pl.kernel(
        _select_body,
        mesh=mesh,
        out_type=[
            jax.ShapeDtypeStruct((_NW, _SLOT), jnp.float32),
            jax.ShapeDtypeStruct((_NW, _SLOT), jnp.int32),
        ],
        scratch_types=[
            pltpu.VMEM((8, _W), jnp.float32),
            pltpu.VMEM((_CAND,), jnp.float32),
            pltpu.VMEM((_CAND,), jnp.int32),
            pltpu.VMEM((_OBUF,), jnp.float32),
            pltpu.VMEM((_OBUF,), jnp.int32),
        ],
        compiler_params=pltpu.CompilerParams(needs_layout_passes=False),
    )
    ov, oi = sel(nm)
    vals, pos = lax.top_k(ov.reshape(-1), _K)
    idxs = oi.reshape(-1)[pos]
    return vals, idxs


_TROW = _H * _W // 16


def _compose_body(lo_hbm, cu_hbm, hi_hbm, idx_hbm, out_hbm,
                  pts_v, idx2d, rows_lo, rows_cu, rows_hi, out_v, sem):
    tid = lax.axis_index("c") * 16 + lax.axis_index("s")
    lanes = lax.iota(jnp.int32, 16)
    zero = jnp.zeros((16,), jnp.float32)

    pltpu.sync_copy(idx_hbm.at[pl.ds(tid * 64, 64)], pts_v)

    for b in range(4):
        p = pts_v[pl.ds(b * 16, 16)]
        y = lax.shift_right_logical(p, 11)
        xm1 = jnp.bitwise_and(p, _W - 1) - 1
        for dy in range(3):
            e0 = (y + (dy - 1)) * _W + xm1
            r0 = lax.shift_right_logical(e0, 4)
            r1 = jnp.minimum(r0 + 1, _TROW - 1)
            cpos = b * 16 + lanes
            plsc.store_scatter(idx2d, [jnp.full((16,), dy * 2, jnp.int32), cpos], r0)
            plsc.store_scatter(idx2d, [jnp.full((16,), dy * 2 + 1, jnp.int32), cpos], r1)

    copies = []
    for tab, rows in ((lo_hbm, rows_lo), (cu_hbm, rows_cu), (hi_hbm, rows_hi)):
        for k in range(6):
            copies.append(pltpu.async_copy(tab.at[idx2d.at[k]], rows.at[k], sem))
    for cp in copies:
        cp.wait()

    for k in range(32):
        out_v[pl.ds(k * 16, 16)] = zero

    inv = 1.0 / float(_W)
    for b in range(4):
        p = pts_v[pl.ds(b * 16, 16)]
        y = lax.shift_right_logical(p, 11)
        xm1 = jnp.bitwise_and(p, _W - 1) - 1
        den = zero
        ns = zero
        ny = zero
        nx = zero
        for dy in range(3):
            e0 = (y + (dy - 1)) * _W + xm1
            r0 = lax.shift_right_logical(e0, 4)
            for d in range(3):
                ee = e0 + d
                o = lax.shift_right_logical(ee, 4) - r0
                d0 = jnp.full((16,), dy * 2, jnp.int32) + o
                d1 = jnp.full((16,), b * 16, jnp.int32) + lanes
                d2 = jnp.bitwise_and(ee, 15)
                for rows, zc in ((rows_lo, -1.0), (rows_cu, 0.0), (rows_hi, 1.0)):
                    val = plsc.load_gather(rows, [d0, d1, d2])
                    den = den + val
                    if zc != 0.0:
                        ns = ns + zc * val
                    if dy != 1:
                        ny = ny + float(dy - 1) * val
                    if d != 1:
                        nx = nx + float(d - 1) * val
        den = den + _EPS_DEN
        s = ns / den * inv
        yc = (ny / den + y.astype(jnp.float32)) * inv
        xc = (nx / den + (xm1 + 1).astype(jnp.float32)) * inv
        rowpos = (jnp.full((16,), b * 16, jnp.int32) + lanes) * 8
        for col, vec in ((0, s), (2, xc), (4, s), (5, yc)):
            plsc.store_scatter(out_v, [rowpos + col], vec)

    pltpu.sync_copy(out_v, out_hbm.at[tid])


def _compose(low2d, cur2d, high2d, idxs):
    mesh = plsc.VectorSubcoreMesh(core_axis_name="c", subcore_axis_name="s")
    comp = pl.kernel(
        _compose_body,
        mesh=mesh,
        out_type=jax.ShapeDtypeStruct((_NW, 512), jnp.float32),
        scratch_types=[
            pltpu.VMEM((64,), jnp.int32),
            pltpu.VMEM((6, 64), jnp.int32),
            pltpu.VMEM((6, 64, 16), jnp.float32),
            pltpu.VMEM((6, 64, 16), jnp.float32),
            pltpu.VMEM((6, 64, 16), jnp.float32),
            pltpu.VMEM((512,), jnp.float32),
            pltpu.SemaphoreType.DMA,
        ],
        compiler_params=pltpu.CompilerParams(needs_layout_passes=False,
                                             use_tc_tiling_on_sc=False),
    )
    idx_pad = jnp.full((_NW * 64,), 2049, jnp.int32).at[:_K].set(idxs)
    out = comp(low2d.reshape(_TROW, 16), cur2d.reshape(_TROW, 16),
               high2d.reshape(_TROW, 16), idx_pad)
    rows6 = out.reshape(_NW * 64, 8)[:_K, :6]
    return rows6.reshape(_K, 2, 3)


def _edges(x):
    zero = jnp.zeros((1, _W), x.dtype)
    up = jnp.concatenate([zero, x[_BLK - 1::_BLK][: _GRID - 1]], axis=0)
    down = jnp.concatenate([x[_BLK::_BLK], zero], axis=0)
    return up.reshape(_GRID, 1, _W), down.reshape(_GRID, 1, _W)


@functools.partial(jax.jit, static_argnums=())
def _run(low, cur, high):
    lo = low.reshape(_H, _W)
    cu = cur.reshape(_H, _W)
    hi = high.reshape(_H, _W)
    lo_u, lo_d = _edges(lo)
    cu_u, cu_d = _edges(cu)
    hi_u, hi_d = _edges(hi)

    blk = pl.BlockSpec((_BLK, _W), lambda i: (i, 0))
    eblk = pl.BlockSpec((1, 1, _W), lambda i: (i, 0, 0))
    out_sd = jax.ShapeDtypeStruct((_H, _W), jnp.float32)
    nm = pl.pallas_call(
        _nms_body,
        grid=(_GRID,),
        in_specs=[blk, blk, blk, eblk, eblk, eblk, eblk, eblk, eblk],
        out_specs=blk,
        out_shape=out_sd,
    )(lo, cu, hi, lo_u, cu_u, hi_u, lo_d, cu_d, hi_d)

    vals, idxs = _select_topk(nm)
    lafs = _compose(lo, cu, hi, idxs)
    return vals, lafs


def kernel(low, cur, high, num_features):
    vals, lafs = _run(low, cur, high)
    return vals, lafs
